# Initial kernel scaffold; baseline (speedup 1.0000x reference)
import jax
import jax.numpy as jnp
from jax.experimental import pallas as pl


def kernel(graphs, x, W0, att_src0, att_dst0, b0, W1, att_src1, att_dst1, b1):
    raise NotImplementedError("write your pallas kernel here")


# speedup vs baseline: 1.0001x; 1.0001x over previous
import functools

import jax
import jax.numpy as jnp
from jax.experimental import pallas as pl


def _mm_body(x_ref, w_ref, o_ref):
    o_ref[...] = jax.lax.dot(
        x_ref[...], w_ref[...],
        precision=jax.lax.Precision.HIGHEST,
        preferred_element_type=jnp.float32,
    )


def _matmul(x, w, bn=1024):
    n, d = x.shape
    _, f = w.shape
    return pl.pallas_call(
        _mm_body,
        grid=(n // bn,),
        in_specs=[
            pl.BlockSpec((bn, d), lambda i: (i, 0)),
            pl.BlockSpec((d, f), lambda i: (0, 0)),
        ],
        out_specs=pl.BlockSpec((bn, f), lambda i: (i, 0)),
        out_shape=jax.ShapeDtypeStruct((n, f), jnp.float32),
    )(x, w)


def _gat_layer(xn, src, dst, W, a_s, a_d, b, concat):
    N = xn.shape[0]
    Hh, C = a_s.shape
    h = _matmul(xn, W).reshape(N, Hh, C)
    a_edge = (h * a_s[None]).sum(-1)[src] + (h * a_d[None]).sum(-1)[dst]
    a_edge = jax.nn.leaky_relu(a_edge, 0.2)
    amax = jax.ops.segment_max(a_edge, dst, num_segments=N)
    ex = jnp.exp(a_edge - amax[dst])
    den = jax.ops.segment_sum(ex, dst, num_segments=N)
    coef = ex / (den[dst] + 1e-16)
    out = jax.ops.segment_sum(h[src] * coef[:, :, None], dst, num_segments=N)
    if concat:
        out = out.reshape(N, Hh * C)
    else:
        out = out.mean(axis=1)
    return out + b


def kernel(graphs, x, W0, att_src0, att_dst0, b0, W1, att_src1, att_dst1, b1):
    Lx, Bx, Dx = x.shape
    N = Bx * Lx
    off = (jnp.arange(Bx, dtype=graphs.dtype) * Lx)[:, None, None]
    ei = graphs + off
    loop = jnp.arange(N, dtype=graphs.dtype)
    src = jnp.concatenate([ei[:, 0, :].reshape(-1), loop])
    dst = jnp.concatenate([ei[:, 1, :].reshape(-1), loop])
    xf = jnp.transpose(x, (1, 0, 2)).reshape(N, Dx)
    h = jax.nn.elu(_gat_layer(xf, src, dst, W0, att_src0, att_dst0, b0, True))
    h = jax.nn.elu(_gat_layer(h, src, dst, W1, att_src1, att_dst1, b1, False))
    return h.reshape(Lx, Bx, Dx)


# speedup vs baseline: 24.6998x; 24.6970x over previous
import dataclasses
import functools

import jax
import jax.numpy as jnp
from jax import lax
from jax.experimental import pallas as pl
from jax.experimental.pallas import tpu as pltpu
from jax.experimental.pallas import tpu_sc as plsc

_L = 2048
_B = 16
_E = 32768
_N = _B * _L
_D = 128
_H = 4

_NC = 2
_NS = 16
_LN = 16
_GPB = _B // _NC
_EPW = _E // _NS
_GRP = _EPW // _LN

_HIGH = jax.lax.Precision.HIGHEST


def _sc_edge_pass(W):
    Wt = W + 16
    NH = W // _H
    mesh = plsc.VectorSubcoreMesh(core_axis_name="c", subcore_axis_name="s")
    cp = pltpu.CompilerParams()
    if "needs_layout_passes" in pltpu.CompilerParams.__dataclass_fields__:
        cp = dataclasses.replace(cp, needs_layout_passes=False)

    @functools.partial(
        pl.kernel,
        out_type=jax.ShapeDtypeStruct((_N * Wt,), jnp.float32),
        mesh=mesh,
        compiler_params=cp,
        scratch_types=[
            pltpu.VMEM_SHARED((_L * Wt,), jnp.float32),
            pltpu.VMEM((_EPW,), jnp.int32),
            pltpu.VMEM((_EPW,), jnp.int32),
            pltpu.VMEM((_L * 8,), jnp.float32),
            pltpu.VMEM((_LN,), jnp.float32),
            pltpu.VMEM((32 * Wt,), jnp.float32),
            pltpu.VMEM((_LN,), jnp.int32),
            pltpu.VMEM((_LN, _D), jnp.float32),
            pltpu.VMEM((_LN * Wt,), jnp.float32),
            pltpu.VMEM((_LN * Wt,), jnp.int32),
            pltpu.SemaphoreType.DMA,
        ],
    )
    def body(feat_hbm, edges_hbm, asad_hbm, m_hbm, out_hbm,
             acc, srcb, dstb, asadb, mb, zbuf, idxb, gbuf, obuf, offb, sem):
        cid = lax.axis_index("c")
        sid = lax.axis_index("s")
        zero16 = jnp.zeros((_LN,), jnp.float32)

        @pl.loop(0, 32 * Wt, step=_LN)
        def _z(j):
            zbuf[pl.ds(j, _LN)] = zero16

        for e in range(_LN):
            obuf[pl.ds(e * Wt + W, _LN)] = zero16

        pltpu.sync_copy(m_hbm, mb)

        @pl.loop(0, _GPB)
        def _graph(g):
            b = cid * _GPB + g
            base = b * _L

            @pl.loop(0, 4)
            def _zacc(z):
                pltpu.sync_copy(
                    zbuf, acc.at[pl.ds((sid * 128 + z * 32) * Wt, 32 * Wt)])

            eoff = b * 2 * _E + sid * _EPW
            pltpu.sync_copy(edges_hbm.at[pl.ds(eoff, _EPW)], srcb)
            pltpu.sync_copy(edges_hbm.at[pl.ds(eoff + _E, _EPW)], dstb)
            pltpu.sync_copy(asad_hbm.at[pl.ds(base * 8, _L * 8)], asadb)
            plsc.subcore_barrier()

            @pl.loop(0, _GRP)
            def _grp(t):
                e0 = t * _LN
                src16 = srcb[pl.ds(e0, _LN)]
                dst16 = dstb[pl.ds(e0, _LN)]
                idxb[...] = src16 + base
                pltpu.async_copy(feat_hbm.at[idxb], gbuf, sem).wait()
                iota16 = lax.iota(jnp.int32, _LN)
                s8 = src16 * 8
                d8 = dst16 * 8
                mvec = mb[...]
                exvecs = []
                for h in range(_H):
                    a_s = plsc.load_gather(asadb, [s8 + h])
                    a_d = plsc.load_gather(asadb, [d8 + (_H + h)])
                    z = a_s + a_d
                    z = jnp.maximum(z, 0.2 * z) - mvec[h]
                    exh = jnp.exp(z)
                    exvecs.append(exh)
                    plsc.store_scatter(
                        obuf, [iota16 * Wt + (W + h)], exh)
                dstW16 = dst16 * Wt
                for e in range(_LN):
                    offv = jnp.full((_LN,), dstW16[e], jnp.int32) + iota16
                    for j in range(Wt // _LN):
                        offb[pl.ds(e * Wt + j * _LN, _LN)] = offv + (j * _LN)
                    chunks = [gbuf[e, pl.ds(j * _LN, _LN)]
                              for j in range(_D // _LN)]
                    for h in range(_H):
                        sv = jnp.full((_LN,), exvecs[h][e], jnp.float32)
                        for j in range(NH // _LN):
                            col = h * NH + j * _LN
                            gsrc = chunks[col // _LN] if W == _D else chunks[j]
                            obuf[pl.ds(e * Wt + col, _LN)] = gsrc * sv
                pltpu.sync_copy(obuf, acc.at[offb], add=True)

            plsc.subcore_barrier()
            pltpu.sync_copy(
                acc.at[pl.ds(sid * 128 * Wt, 128 * Wt)],
                out_hbm.at[pl.ds((base + sid * 128) * Wt, 128 * Wt)])

    return body


_sc_pass_l0 = _sc_edge_pass(128)
_sc_pass_l1 = _sc_edge_pass(512)


_BN = 2048


def _pre0_body(x_ref, w_ref, p_ref, h_ref, a_ref, m_ref):
    i = pl.program_id(0)
    h = lax.dot(x_ref[...], w_ref[...], precision=_HIGH,
                preferred_element_type=jnp.float32)
    h_ref[...] = h
    a = lax.dot(h, p_ref[...], precision=_HIGH,
                preferred_element_type=jnp.float32)
    a_ref[...] = a
    cur = jnp.broadcast_to(jnp.max(a, axis=0)[:, None], (8, 128))

    @pl.when(i == 0)
    def _init():
        m_ref[...] = cur

    @pl.when(i > 0)
    def _acc():
        m_ref[...] = jnp.maximum(m_ref[...], cur)


def _pre1_body(x_ref, v_ref, a_ref, m_ref):
    i = pl.program_id(0)
    a = lax.dot(x_ref[...], v_ref[...], precision=_HIGH,
                preferred_element_type=jnp.float32)
    a_ref[...] = a
    cur = jnp.broadcast_to(jnp.max(a, axis=0)[:, None], (8, 128))

    @pl.when(i == 0)
    def _init():
        m_ref[...] = cur

    @pl.when(i > 0)
    def _acc():
        m_ref[...] = jnp.maximum(m_ref[...], cur)


def _elu(o):
    return jnp.where(o > 0.0, o, jnp.exp(o) - 1.0)


def _post0_body(u_ref, h_ref, a_ref, m_ref, b_ref, o_ref):
    a = a_ref[...]
    u = u_ref[...]
    h = h_ref[...]
    z = a[:, :4] + a[:, 4:8]
    z = jnp.maximum(z, 0.2 * z)
    exs = jnp.exp(z - m_ref[...][:, 0:4])
    den = u[:, 128:132] + exs + 1e-16
    parts = []
    for hh in range(_H):
        sl = slice(hh * 32, (hh + 1) * 32)
        num = u[:, sl] + exs[:, hh:hh + 1] * h[:, sl]
        parts.append(num / den[:, hh:hh + 1])
    o = jnp.concatenate(parts, axis=1) + b_ref[...]
    o_ref[...] = _elu(o)


def _post1_body(u_ref, x_ref, a_ref, m_ref, w_ref, b_ref, o_ref):
    a = a_ref[...]
    u = u_ref[...]
    x1 = x_ref[...]
    z = a[:, :4] + a[:, 4:8]
    z = jnp.maximum(z, 0.2 * z)
    exs = jnp.exp(z - m_ref[...][:, 0:4])
    den = u[:, 512:516] + exs + 1e-16
    parts = []
    for hh in range(_H):
        sl = slice(hh * 128, (hh + 1) * 128)
        num = u[:, sl] + exs[:, hh:hh + 1] * x1
        parts.append(num / den[:, hh:hh + 1])
    agg = jnp.concatenate(parts, axis=1) * 0.25
    o = lax.dot(agg, w_ref[...], precision=_HIGH,
                preferred_element_type=jnp.float32) + b_ref[...]
    o_ref[...] = _elu(o)


def _pre0(xf, W0, P0):
    return pl.pallas_call(
        _pre0_body,
        grid=(_N // _BN,),
        in_specs=[
            pl.BlockSpec((_BN, _D), lambda i: (i, 0)),
            pl.BlockSpec((_D, _D), lambda i: (0, 0)),
            pl.BlockSpec((_D, 8), lambda i: (0, 0)),
        ],
        out_specs=[
            pl.BlockSpec((_BN, _D), lambda i: (i, 0)),
            pl.BlockSpec((_BN, 8), lambda i: (i, 0)),
            pl.BlockSpec((8, 128), lambda i: (0, 0)),
        ],
        out_shape=[
            jax.ShapeDtypeStruct((_N, _D), jnp.float32),
            jax.ShapeDtypeStruct((_N, 8), jnp.float32),
            jax.ShapeDtypeStruct((8, 128), jnp.float32),
        ],
    )(xf, W0, P0)


def _pre1(x1, Vsd1):
    return pl.pallas_call(
        _pre1_body,
        grid=(_N // _BN,),
        in_specs=[
            pl.BlockSpec((_BN, _D), lambda i: (i, 0)),
            pl.BlockSpec((_D, 8), lambda i: (0, 0)),
        ],
        out_specs=[
            pl.BlockSpec((_BN, 8), lambda i: (i, 0)),
            pl.BlockSpec((8, 128), lambda i: (0, 0)),
        ],
        out_shape=[
            jax.ShapeDtypeStruct((_N, 8), jnp.float32),
            jax.ShapeDtypeStruct((8, 128), jnp.float32),
        ],
    )(x1, Vsd1)


def _post0(uagg, h0, asad, m128, b2d):
    return pl.pallas_call(
        _post0_body,
        grid=(_N // _BN,),
        in_specs=[
            pl.BlockSpec((_BN, 144), lambda i: (i, 0)),
            pl.BlockSpec((_BN, _D), lambda i: (i, 0)),
            pl.BlockSpec((_BN, 8), lambda i: (i, 0)),
            pl.BlockSpec((1, 128), lambda i: (0, 0)),
            pl.BlockSpec((1, 128), lambda i: (0, 0)),
        ],
        out_specs=pl.BlockSpec((_BN, _D), lambda i: (i, 0)),
        out_shape=jax.ShapeDtypeStruct((_N, _D), jnp.float32),
    )(uagg, h0, asad, m128, b2d)


def _post1(uagg, x1, asad, m128, W1r, b2d):
    return pl.pallas_call(
        _post1_body,
        grid=(_N // _BN,),
        in_specs=[
            pl.BlockSpec((_BN, 528), lambda i: (i, 0)),
            pl.BlockSpec((_BN, _D), lambda i: (i, 0)),
            pl.BlockSpec((_BN, 8), lambda i: (i, 0)),
            pl.BlockSpec((1, 128), lambda i: (0, 0)),
            pl.BlockSpec((512, 128), lambda i: (0, 0)),
            pl.BlockSpec((1, 128), lambda i: (0, 0)),
        ],
        out_specs=pl.BlockSpec((_BN, _D), lambda i: (i, 0)),
        out_shape=jax.ShapeDtypeStruct((_N, _D), jnp.float32),
    )(uagg, x1, asad, m128, W1r, b2d)


def _bounds(mx):
    mvec = mx[:, 0]
    s = mvec[:4] + mvec[4:]
    m4 = jnp.maximum(s, 0.2 * s)
    m8 = jnp.concatenate([m4, jnp.zeros((12,), jnp.float32)])
    m128 = jnp.zeros((1, 128), jnp.float32).at[0, :4].set(m4)
    return m8, m128


def kernel(graphs, x, W0, att_src0, att_dst0, b0, W1, att_src1, att_dst1, b1):
    Lx, Bx, Dx = x.shape
    xf = jnp.transpose(x, (1, 0, 2)).reshape(_N, _D)
    edges_flat = graphs.astype(jnp.int32).reshape(-1)

    eye4 = jnp.eye(4, dtype=jnp.float32)
    P0 = jnp.concatenate(
        [att_src0[:, :, None] * eye4[:, None, :],
         att_dst0[:, :, None] * eye4[:, None, :]], axis=2).reshape(_D, 8)
    W1h = W1.reshape(_D, _H, _D)
    Vsd1 = jnp.concatenate(
        [jnp.einsum('dhc,hc->dh', W1h, att_src1),
         jnp.einsum('dhc,hc->dh', W1h, att_dst1)], axis=1)
    W1r = W1h.transpose(1, 0, 2).reshape(_H * _D, _D)

    h0, asad0, mx0 = _pre0(xf, W0, P0)
    m8_0, m128_0 = _bounds(mx0)
    uagg0 = _sc_pass_l0(h0, edges_flat, asad0.reshape(-1), m8_0)
    x1 = _post0(uagg0.reshape(_N, 144), h0, asad0, m128_0, b0.reshape(1, _D))

    asad1, mx1 = _pre1(x1, Vsd1)
    m8_1, m128_1 = _bounds(mx1)
    uagg1 = _sc_pass_l1(x1, edges_flat, asad1.reshape(-1), m8_1)
    out = _post1(uagg1.reshape(_N, 528), x1, asad1, m128_1, W1r,
                 b1.reshape(1, _D))

    return out.reshape(Lx, Bx, Dx)


# speedup vs baseline: 38.8628x; 1.5734x over previous
import dataclasses
import functools

import jax
import jax.numpy as jnp
from jax import lax
from jax.experimental import pallas as pl
from jax.experimental.pallas import tpu as pltpu
from jax.experimental.pallas import tpu_sc as plsc

_L = 2048
_B = 16
_E = 32768
_N = _B * _L
_D = 128
_H = 4

_NC = 2
_NS = 16
_LN = 16
_GPB = _B // _NC
_EPW = _E // _NS
_GRP = _EPW // _LN

_HIGH = jax.lax.Precision.HIGHEST


def _sc_edge_pass(W):
    Wt = W + 16
    NH = W // _H
    mesh = plsc.VectorSubcoreMesh(core_axis_name="c", subcore_axis_name="s")
    cp = pltpu.CompilerParams()
    if "needs_layout_passes" in pltpu.CompilerParams.__dataclass_fields__:
        cp = dataclasses.replace(cp, needs_layout_passes=False)

    @functools.partial(
        pl.kernel,
        out_type=jax.ShapeDtypeStruct((_N * Wt,), jnp.float32),
        mesh=mesh,
        compiler_params=cp,
        scratch_types=[
            pltpu.VMEM_SHARED((_L * Wt,), jnp.float32),
            pltpu.VMEM((_EPW,), jnp.int32),
            pltpu.VMEM((_EPW,), jnp.int32),
            pltpu.VMEM((_L * 8,), jnp.float32),
            pltpu.VMEM((_LN,), jnp.float32),
            pltpu.VMEM((8 * Wt,), jnp.float32),
            pltpu.VMEM((_LN,), jnp.int32),
            pltpu.VMEM((_LN,), jnp.int32),
            pltpu.VMEM((_LN, _D), jnp.float32),
            pltpu.VMEM((_LN, _D), jnp.float32),
            pltpu.VMEM((_LN * Wt,), jnp.float32),
            pltpu.VMEM((_LN * Wt,), jnp.float32),
            pltpu.VMEM((_LN * Wt,), jnp.int32),
            pltpu.VMEM((_LN * Wt,), jnp.int32),
            pltpu.SemaphoreType.DMA,
            pltpu.SemaphoreType.DMA,
            pltpu.SemaphoreType.DMA,
            pltpu.SemaphoreType.DMA,
        ],
    )
    def body(feat_hbm, edges_hbm, asad_hbm, m_hbm, out_hbm,
             acc, srcb, dstb, asadb, mb, zbuf, idxb0, idxb1, gbuf0, gbuf1,
             obuf0, obuf1, offb0, offb1, gsem0, gsem1, ssem0, ssem1):
        idxbs = (idxb0, idxb1)
        gbufs = (gbuf0, gbuf1)
        obufs = (obuf0, obuf1)
        offbs = (offb0, offb1)
        gsems = (gsem0, gsem1)
        ssems = (ssem0, ssem1)
        cid = lax.axis_index("c")
        sid = lax.axis_index("s")
        zero16 = jnp.zeros((_LN,), jnp.float32)

        @pl.loop(0, 8 * Wt, step=_LN)
        def _z(j):
            zbuf[pl.ds(j, _LN)] = zero16

        for e in range(_LN):
            obuf0[pl.ds(e * Wt + W, _LN)] = zero16
            obuf1[pl.ds(e * Wt + W, _LN)] = zero16

        pltpu.sync_copy(m_hbm, mb)

        @pl.loop(0, _GPB)
        def _graph(g):
            b = cid * _GPB + g
            base = b * _L

            @pl.loop(0, 16)
            def _zacc(z):
                pltpu.sync_copy(
                    zbuf, acc.at[pl.ds((sid * 128 + z * 8) * Wt, 8 * Wt)])

            eoff = b * 2 * _E + sid * _EPW
            pltpu.sync_copy(edges_hbm.at[pl.ds(eoff, _EPW)], srcb)
            pltpu.sync_copy(edges_hbm.at[pl.ds(eoff + _E, _EPW)], dstb)
            pltpu.sync_copy(asad_hbm.at[pl.ds(base * 8, _L * 8)], asadb)
            plsc.subcore_barrier()

            idxbs[0][...] = srcb[pl.ds(0, _LN)] + base
            pltpu.async_copy(feat_hbm.at[idxbs[0]], gbufs[0], gsems[0])

            @pl.loop(0, _GRP, step=2)
            def _grp(t):
                for k in range(2):
                    g = t + k
                    gb, ob, fb = gbufs[k], obufs[k], offbs[k]

                    @pl.when(t > 0)
                    def _wait_prev_scatter():
                        pltpu.make_async_copy(ob, acc.at[fb], ssems[k]).wait()

                    pltpu.make_async_copy(
                        feat_hbm.at[idxbs[k]], gb, gsems[k]).wait()

                    kn = 1 - k
                    if k == 0:
                        idxbs[kn][...] = srcb[pl.ds((g + 1) * _LN, _LN)] + base
                        pltpu.async_copy(
                            feat_hbm.at[idxbs[kn]], gbufs[kn], gsems[kn])
                    else:
                        @pl.when(t < _GRP - 2)
                        def _prefetch():
                            idxbs[kn][...] = (
                                srcb[pl.ds((g + 1) * _LN, _LN)] + base)
                            pltpu.async_copy(
                                feat_hbm.at[idxbs[kn]], gbufs[kn], gsems[kn])

                    e0 = g * _LN
                    src16 = srcb[pl.ds(e0, _LN)]
                    dst16 = dstb[pl.ds(e0, _LN)]
                    iota16 = lax.iota(jnp.int32, _LN)
                    s8 = src16 * 8
                    d8 = dst16 * 8
                    mvec = mb[...]
                    exvecs = []
                    for h in range(_H):
                        a_s = plsc.load_gather(asadb, [s8 + h])
                        a_d = plsc.load_gather(asadb, [d8 + (_H + h)])
                        z = a_s + a_d
                        z = jnp.maximum(z, 0.2 * z) - mvec[h]
                        exh = jnp.exp(z)
                        exvecs.append(exh)
                        plsc.store_scatter(ob, [iota16 * Wt + (W + h)], exh)
                    dstW16 = dst16 * Wt
                    for e in range(_LN):
                        offv = jnp.full((_LN,), dstW16[e], jnp.int32) + iota16
                        for j in range(Wt // _LN):
                            fb[pl.ds(e * Wt + j * _LN, _LN)] = offv + (j * _LN)
                        chunks = [gb[e, pl.ds(j * _LN, _LN)]
                                  for j in range(_D // _LN)]
                        for h in range(_H):
                            sv = jnp.full((_LN,), exvecs[h][e], jnp.float32)
                            for j in range(NH // _LN):
                                col = h * NH + j * _LN
                                gsrc = (chunks[col // _LN] if W == _D
                                        else chunks[j])
                                ob[pl.ds(e * Wt + col, _LN)] = gsrc * sv
                    pltpu.async_copy(ob, acc.at[fb], ssems[k], add=True)

            pltpu.make_async_copy(obufs[0], acc.at[offbs[0]], ssems[0]).wait()
            pltpu.make_async_copy(obufs[1], acc.at[offbs[1]], ssems[1]).wait()
            plsc.subcore_barrier()
            pltpu.sync_copy(
                acc.at[pl.ds(sid * 128 * Wt, 128 * Wt)],
                out_hbm.at[pl.ds((base + sid * 128) * Wt, 128 * Wt)])

    return body


_sc_pass_l0 = _sc_edge_pass(128)
_sc_pass_l1 = _sc_edge_pass(512)


_BN = 2048


def _pre0_body(x_ref, w_ref, p_ref, h_ref, a_ref, m_ref):
    i = pl.program_id(0)
    h = lax.dot(x_ref[...], w_ref[...], precision=_HIGH,
                preferred_element_type=jnp.float32)
    h_ref[...] = h
    a = lax.dot(h, p_ref[...], precision=_HIGH,
                preferred_element_type=jnp.float32)
    a_ref[...] = a
    cur = jnp.broadcast_to(jnp.max(a, axis=0)[:, None], (8, 128))

    @pl.when(i == 0)
    def _init():
        m_ref[...] = cur

    @pl.when(i > 0)
    def _acc():
        m_ref[...] = jnp.maximum(m_ref[...], cur)


def _pre1_body(x_ref, v_ref, a_ref, m_ref):
    i = pl.program_id(0)
    a = lax.dot(x_ref[...], v_ref[...], precision=_HIGH,
                preferred_element_type=jnp.float32)
    a_ref[...] = a
    cur = jnp.broadcast_to(jnp.max(a, axis=0)[:, None], (8, 128))

    @pl.when(i == 0)
    def _init():
        m_ref[...] = cur

    @pl.when(i > 0)
    def _acc():
        m_ref[...] = jnp.maximum(m_ref[...], cur)


def _elu(o):
    return jnp.where(o > 0.0, o, jnp.exp(o) - 1.0)


def _post0_body(u_ref, h_ref, a_ref, m_ref, b_ref, o_ref):
    a = a_ref[...]
    u = u_ref[...]
    h = h_ref[...]
    z = a[:, :4] + a[:, 4:8]
    z = jnp.maximum(z, 0.2 * z)
    exs = jnp.exp(z - m_ref[...][:, 0:4])
    den = u[:, 128:132] + exs + 1e-16
    parts = []
    for hh in range(_H):
        sl = slice(hh * 32, (hh + 1) * 32)
        num = u[:, sl] + exs[:, hh:hh + 1] * h[:, sl]
        parts.append(num / den[:, hh:hh + 1])
    o = jnp.concatenate(parts, axis=1) + b_ref[...]
    o_ref[...] = _elu(o)


def _post1_body(u_ref, x_ref, a_ref, m_ref, w_ref, b_ref, o_ref):
    a = a_ref[...]
    u = u_ref[...]
    x1 = x_ref[...]
    z = a[:, :4] + a[:, 4:8]
    z = jnp.maximum(z, 0.2 * z)
    exs = jnp.exp(z - m_ref[...][:, 0:4])
    den = u[:, 512:516] + exs + 1e-16
    parts = []
    for hh in range(_H):
        sl = slice(hh * 128, (hh + 1) * 128)
        num = u[:, sl] + exs[:, hh:hh + 1] * x1
        parts.append(num / den[:, hh:hh + 1])
    agg = jnp.concatenate(parts, axis=1) * 0.25
    o = lax.dot(agg, w_ref[...], precision=_HIGH,
                preferred_element_type=jnp.float32) + b_ref[...]
    o_ref[...] = _elu(o)


def _pre0(xf, W0, P0):
    return pl.pallas_call(
        _pre0_body,
        grid=(_N // _BN,),
        in_specs=[
            pl.BlockSpec((_BN, _D), lambda i: (i, 0)),
            pl.BlockSpec((_D, _D), lambda i: (0, 0)),
            pl.BlockSpec((_D, 8), lambda i: (0, 0)),
        ],
        out_specs=[
            pl.BlockSpec((_BN, _D), lambda i: (i, 0)),
            pl.BlockSpec((_BN, 8), lambda i: (i, 0)),
            pl.BlockSpec((8, 128), lambda i: (0, 0)),
        ],
        out_shape=[
            jax.ShapeDtypeStruct((_N, _D), jnp.float32),
            jax.ShapeDtypeStruct((_N, 8), jnp.float32),
            jax.ShapeDtypeStruct((8, 128), jnp.float32),
        ],
    )(xf, W0, P0)


def _pre1(x1, Vsd1):
    return pl.pallas_call(
        _pre1_body,
        grid=(_N // _BN,),
        in_specs=[
            pl.BlockSpec((_BN, _D), lambda i: (i, 0)),
            pl.BlockSpec((_D, 8), lambda i: (0, 0)),
        ],
        out_specs=[
            pl.BlockSpec((_BN, 8), lambda i: (i, 0)),
            pl.BlockSpec((8, 128), lambda i: (0, 0)),
        ],
        out_shape=[
            jax.ShapeDtypeStruct((_N, 8), jnp.float32),
            jax.ShapeDtypeStruct((8, 128), jnp.float32),
        ],
    )(x1, Vsd1)


def _post0(uagg, h0, asad, m128, b2d):
    return pl.pallas_call(
        _post0_body,
        grid=(_N // _BN,),
        in_specs=[
            pl.BlockSpec((_BN, 144), lambda i: (i, 0)),
            pl.BlockSpec((_BN, _D), lambda i: (i, 0)),
            pl.BlockSpec((_BN, 8), lambda i: (i, 0)),
            pl.BlockSpec((1, 128), lambda i: (0, 0)),
            pl.BlockSpec((1, 128), lambda i: (0, 0)),
        ],
        out_specs=pl.BlockSpec((_BN, _D), lambda i: (i, 0)),
        out_shape=jax.ShapeDtypeStruct((_N, _D), jnp.float32),
    )(uagg, h0, asad, m128, b2d)


def _post1(uagg, x1, asad, m128, W1r, b2d):
    return pl.pallas_call(
        _post1_body,
        grid=(_N // _BN,),
        in_specs=[
            pl.BlockSpec((_BN, 528), lambda i: (i, 0)),
            pl.BlockSpec((_BN, _D), lambda i: (i, 0)),
            pl.BlockSpec((_BN, 8), lambda i: (i, 0)),
            pl.BlockSpec((1, 128), lambda i: (0, 0)),
            pl.BlockSpec((512, 128), lambda i: (0, 0)),
            pl.BlockSpec((1, 128), lambda i: (0, 0)),
        ],
        out_specs=pl.BlockSpec((_BN, _D), lambda i: (i, 0)),
        out_shape=jax.ShapeDtypeStruct((_N, _D), jnp.float32),
    )(uagg, x1, asad, m128, W1r, b2d)


def _bounds(mx):
    mvec = mx[:, 0]
    s = mvec[:4] + mvec[4:]
    m4 = jnp.maximum(s, 0.2 * s)
    m8 = jnp.concatenate([m4, jnp.zeros((12,), jnp.float32)])
    m128 = jnp.zeros((1, 128), jnp.float32).at[0, :4].set(m4)
    return m8, m128


def kernel(graphs, x, W0, att_src0, att_dst0, b0, W1, att_src1, att_dst1, b1):
    Lx, Bx, Dx = x.shape
    xf = jnp.transpose(x, (1, 0, 2)).reshape(_N, _D)
    edges_flat = graphs.astype(jnp.int32).reshape(-1)

    eye4 = jnp.eye(4, dtype=jnp.float32)
    P0 = jnp.concatenate(
        [att_src0[:, :, None] * eye4[:, None, :],
         att_dst0[:, :, None] * eye4[:, None, :]], axis=2).reshape(_D, 8)
    W1h = W1.reshape(_D, _H, _D)
    Vsd1 = jnp.concatenate(
        [jnp.einsum('dhc,hc->dh', W1h, att_src1),
         jnp.einsum('dhc,hc->dh', W1h, att_dst1)], axis=1)
    W1r = W1h.transpose(1, 0, 2).reshape(_H * _D, _D)

    h0, asad0, mx0 = _pre0(xf, W0, P0)
    m8_0, m128_0 = _bounds(mx0)
    uagg0 = _sc_pass_l0(h0, edges_flat, asad0.reshape(-1), m8_0)
    x1 = _post0(uagg0.reshape(_N, 144), h0, asad0, m128_0, b0.reshape(1, _D))

    asad1, mx1 = _pre1(x1, Vsd1)
    m8_1, m128_1 = _bounds(mx1)
    uagg1 = _sc_pass_l1(x1, edges_flat, asad1.reshape(-1), m8_1)
    out = _post1(uagg1.reshape(_N, 528), x1, asad1, m128_1, W1r,
                 b1.reshape(1, _D))

    return out.reshape(Lx, Bx, Dx)
